# Initial kernel scaffold; baseline (speedup 1.0000x reference)
import jax
import jax.numpy as jnp
from jax.experimental import pallas as pl


def kernel(x, edge_index, batch, W1, b1, W2, b2, W3, b3, W4, b4, Wfc, bfc):
    raise NotImplementedError("write your pallas kernel here")


# speedup vs baseline: 11.6351x; 11.6351x over previous
import functools

import jax
import jax.numpy as jnp
from jax import lax
from jax.experimental import pallas as pl
from jax.experimental.pallas import tpu as pltpu
from jax.experimental.pallas import tpu_sc as plsc

_N = 10000
_E = 320000
_G = 64
_NC = 2
_NS = 16
_NW = _NC * _NS
_EPW = _E // _NW
_C = 80
_NCH = _EPW // _C
_RCH = _N // _C
_RT = (_RCH + _NS - 1) // _NS


def _sc_mesh():
    return plsc.VectorSubcoreMesh(core_axis_name="c", subcore_axis_name="s")


def _make_degree_kernel():
    def body(col_hbm, out_hbm, cidx_v, ones_v, zeros_v, acc_sh):
        c = lax.axis_index("c")
        s = lax.axis_index("s")
        wid = c * _NS + s

        ones16 = jnp.ones((16,), jnp.float32)
        zero16 = jnp.zeros((16,), jnp.float32)

        def fill(k, _):
            ones_v[0, pl.ds(k * 16, 16)] = ones16
            zeros_v[pl.ds(k * 16, 16)] = zero16
            return 0

        lax.fori_loop(0, _C // 16, fill, 0)

        def zinit(t, _):
            j = s + t * _NS

            @pl.when(j < _RCH)
            def _():
                pltpu.sync_copy(zeros_v, acc_sh.at[pl.ds(j * _C, _C)])

            return 0

        lax.fori_loop(0, _RT, zinit, 0)
        plsc.subcore_barrier()

        base0 = wid * _EPW

        def step(j, _):
            b = base0 + j * _C
            pltpu.sync_copy(col_hbm.at[pl.ds(b, _C)], cidx_v.at[0])
            pltpu.sync_copy(ones_v.at[0], acc_sh.at[cidx_v.at[0]], add=True)
            return 0

        lax.fori_loop(0, _NCH, step, 0)
        plsc.subcore_barrier()

        def flush(t, _):
            j = s + t * _NS

            @pl.when(j < _RCH)
            def _():
                pltpu.sync_copy(acc_sh.at[pl.ds(j * _C, _C)], zeros_v)
                pltpu.sync_copy(zeros_v, out_hbm.at[pl.ds(c * _N + j * _C, _C)])

            return 0

        lax.fori_loop(0, _RT, flush, 0)

    return pl.kernel(
        body,
        out_type=jax.ShapeDtypeStruct((_NC * _N,), jnp.float32),
        mesh=_sc_mesh(),
        scratch_types=[
            pltpu.VMEM((1, _C), jnp.int32),
            pltpu.VMEM((1, _C), jnp.float32),
            pltpu.VMEM((_C,), jnp.float32),
            pltpu.VMEM_SHARED((_N,), jnp.float32),
        ],
    )


def _make_scatter_kernel(D):
    def body(xs_hbm, row_hbm, col_hbm, out_hbm, idx_v, msg_v, acc_sh, sem):
        c = lax.axis_index("c")
        s = lax.axis_index("s")
        wid = c * _NS + s

        zero16 = jnp.zeros((16,), jnp.float32)

        def zrow(r, _):
            def zcol(k, _):
                msg_v[r, pl.ds(k * 16, 16)] = zero16
                return 0

            lax.fori_loop(0, D // 16, zcol, 0)
            return 0

        lax.fori_loop(0, _C, zrow, 0)

        def zinit(t, _):
            j = s + t * _NS

            @pl.when(j < _RCH)
            def _():
                pltpu.sync_copy(msg_v, acc_sh.at[pl.ds(j * _C, _C)])

            return 0

        lax.fori_loop(0, _RT, zinit, 0)
        plsc.subcore_barrier()

        base0 = wid * _EPW

        def step(j, _):
            b = base0 + j * _C
            pltpu.sync_copy(row_hbm.at[pl.ds(b, _C)], idx_v.at[0])
            pltpu.sync_copy(col_hbm.at[pl.ds(b, _C)], idx_v.at[1])
            pltpu.async_copy(xs_hbm.at[idx_v.at[0]], msg_v, sem).wait()
            pltpu.sync_copy(msg_v, acc_sh.at[idx_v.at[1]], add=True)
            return 0

        lax.fori_loop(0, _NCH, step, 0)
        plsc.subcore_barrier()

        def flush(t, _):
            j = s + t * _NS

            @pl.when(j < _RCH)
            def _():
                pltpu.sync_copy(acc_sh.at[pl.ds(j * _C, _C)], msg_v)
                pltpu.sync_copy(msg_v, out_hbm.at[pl.ds(c * _N + j * _C, _C)])

            return 0

        lax.fori_loop(0, _RT, flush, 0)

    return pl.kernel(
        body,
        out_type=jax.ShapeDtypeStruct((_NC * _N, D), jnp.float32),
        mesh=_sc_mesh(),
        compiler_params=pltpu.CompilerParams(use_tc_tiling_on_sc=False),
        scratch_types=[
            pltpu.VMEM((2, _C), jnp.int32),
            pltpu.VMEM((_C, D), jnp.float32),
            pltpu.VMEM_SHARED((_N, D), jnp.float32),
            pltpu.SemaphoreType.DMA,
        ],
    )


def _tc_first(x, W, degp):
    def body(x_ref, w_ref, d_ref, xs_ref, dis_ref):
        deg = d_ref[:_N, :] + d_ref[_N:, :] + 1.0
        dis = lax.rsqrt(deg)
        dis_ref[...] = dis
        xs_ref[...] = jnp.dot(x_ref[...], w_ref[...],
                              preferred_element_type=jnp.float32) * dis

    return pl.pallas_call(
        body,
        out_shape=(
            jax.ShapeDtypeStruct((_N, W.shape[1]), jnp.float32),
            jax.ShapeDtypeStruct((_N, 1), jnp.float32),
        ),
    )(x, W, degp)


def _tc_fuse(scat, xs, dis, b, W):
    def body(sc_ref, xs_ref, dis_ref, b_ref, w_ref, o_ref):
        h = sc_ref[:_N, :] + sc_ref[_N:, :] + xs_ref[...]
        h = jnp.maximum(h * dis_ref[...] + b_ref[...], 0.0)
        o_ref[...] = jnp.dot(h, w_ref[...],
                             preferred_element_type=jnp.float32) * dis_ref[...]

    return pl.pallas_call(
        body,
        out_shape=jax.ShapeDtypeStruct((_N, W.shape[1]), jnp.float32),
    )(scat, xs, dis, b, W)


def _tc_final(scat, xs, dis, b, batch_row, Wfc, bfc):
    def body(sc_ref, xs_ref, dis_ref, b_ref, bt_ref, wfc_ref, bfc_ref, o_ref):
        h = sc_ref[:_N, :] + sc_ref[_N:, :] + xs_ref[...]
        h = jnp.maximum(h * dis_ref[...] + b_ref[...], 0.0)
        gids = lax.broadcasted_iota(jnp.int32, (_G, _N), 0)
        m = (bt_ref[...] == gids).astype(jnp.float32)
        sums = jnp.dot(m, h, preferred_element_type=jnp.float32)
        cnt = jnp.maximum(jnp.sum(m, axis=1), 1.0)
        pooled = sums / cnt[:, None]
        o_ref[...] = jnp.dot(pooled, wfc_ref[...],
                             preferred_element_type=jnp.float32) + bfc_ref[...]

    return pl.pallas_call(
        body,
        out_shape=jax.ShapeDtypeStruct((_G, 1), jnp.float32),
    )(scat, xs, dis, b, batch_row, Wfc, bfc)


def kernel(x, edge_index, batch, W1, b1, W2, b2, W3, b3, W4, b4, Wfc, bfc):
    ei = edge_index.astype(jnp.int32)
    row, col = ei[0], ei[1]

    degp = _make_degree_kernel()(col)
    xs1, dis = _tc_first(x, W1, degp.reshape(_NC * _N, 1))

    sc1 = _make_scatter_kernel(128)(xs1, row, col)
    xs2 = _tc_fuse(sc1, xs1, dis, b1.reshape(1, -1), W2)

    sc2 = _make_scatter_kernel(64)(xs2, row, col)
    xs3 = _tc_fuse(sc2, xs2, dis, b2.reshape(1, -1), W3)

    sc3 = _make_scatter_kernel(64)(xs3, row, col)
    xs4 = _tc_fuse(sc3, xs3, dis, b3.reshape(1, -1), W4)

    sc4 = _make_scatter_kernel(32)(xs4, row, col)
    return _tc_final(sc4, xs4, dis, b4.reshape(1, -1),
                     batch.reshape(1, _N).astype(jnp.int32),
                     Wfc, bfc.reshape(1, 1))


# speedup vs baseline: 37.7082x; 3.2409x over previous
import functools

import jax
import jax.numpy as jnp
from jax import lax
from jax.experimental import pallas as pl
from jax.experimental.pallas import tpu as pltpu
from jax.experimental.pallas import tpu_sc as plsc

_N = 10000
_E = 320000
_G = 64
_NC = 2
_NS = 16
_NW = _NC * _NS
_EPW = _E // _NW
_C = 80
_NCH = _EPW // _C
_NB = 5
_RCH = _N // _C
_RT = (_RCH + _NS - 1) // _NS

_SC_PARAMS = pltpu.CompilerParams(use_tc_tiling_on_sc=False)


def _sc_mesh():
    return plsc.VectorSubcoreMesh(core_axis_name="c", subcore_axis_name="s")


def _make_degree_kernel():
    def body(col_hbm, out_hbm, cidx_v, ones_v, zeros_v, acc_sh):
        c = lax.axis_index("c")
        s = lax.axis_index("s")
        wid = c * _NS + s

        ones16 = jnp.ones((16,), jnp.float32)
        zero16 = jnp.zeros((16,), jnp.float32)

        def fill(k, _):
            ones_v[0, pl.ds(k * 16, 16)] = ones16
            zeros_v[pl.ds(k * 16, 16)] = zero16
            return 0

        lax.fori_loop(0, _C // 16, fill, 0)

        def zinit(t, _):
            j = s + t * _NS

            @pl.when(j < _RCH)
            def _():
                pltpu.sync_copy(zeros_v, acc_sh.at[pl.ds(j * _C, _C)])

            return 0

        lax.fori_loop(0, _RT, zinit, 0)

        pltpu.sync_copy(col_hbm.at[pl.ds(wid * _NCH, _NCH)], cidx_v)
        plsc.subcore_barrier()

        def step(j, _):
            pltpu.sync_copy(ones_v.at[0], acc_sh.at[cidx_v.at[j]], add=True)
            return 0

        lax.fori_loop(0, _NCH, step, 0)
        plsc.subcore_barrier()

        def flush(t, _):
            j = s + t * _NS

            @pl.when(j < _RCH)
            def _():
                pltpu.sync_copy(acc_sh.at[pl.ds(j * _C, _C)], zeros_v)
                pltpu.sync_copy(zeros_v, out_hbm.at[pl.ds(c * _N + j * _C, _C)])

            return 0

        lax.fori_loop(0, _RT, flush, 0)

    return pl.kernel(
        body,
        out_type=jax.ShapeDtypeStruct((_NC * _N,), jnp.float32),
        mesh=_sc_mesh(),
        scratch_types=[
            pltpu.VMEM((_NCH, _C), jnp.int32),
            pltpu.VMEM((1, _C), jnp.float32),
            pltpu.VMEM((_C,), jnp.float32),
            pltpu.VMEM_SHARED((_N,), jnp.float32),
        ],
        compiler_params=_SC_PARAMS,
    )


def _make_scatter_kernel(D, C, NB):
    NCH = _EPW // C
    RCH = _N // C
    RT = (RCH + _NS - 1) // _NS

    def body(xs_hbm, row_hbm, col_hbm, out_hbm,
             ridx_v, cidx_v, *bufs_and_sems):
        bufs = bufs_and_sems[:NB]
        acc_sh = bufs_and_sems[NB]
        gsems = bufs_and_sems[NB + 1:]
        c = lax.axis_index("c")
        s = lax.axis_index("s")
        wid = c * _NS + s

        zero16 = jnp.zeros((16,), jnp.float32)

        def zrow(r, _):
            def zcol(k, _):
                bufs[0][r, pl.ds(k * 16, 16)] = zero16
                return 0

            lax.fori_loop(0, D // 16, zcol, 0)
            return 0

        lax.fori_loop(0, C, zrow, 0)

        def zinit(t, _):
            j = s + t * _NS

            @pl.when(j < RCH)
            def _():
                pltpu.sync_copy(bufs[0], acc_sh.at[pl.ds(j * C, C)])

            return 0

        lax.fori_loop(0, RT, zinit, 0)

        pltpu.sync_copy(row_hbm.at[pl.ds(wid * NCH, NCH)], ridx_v)
        pltpu.sync_copy(col_hbm.at[pl.ds(wid * NCH, NCH)], cidx_v)
        plsc.subcore_barrier()

        for b in range(NB):
            pltpu.async_copy(xs_hbm.at[ridx_v.at[b]], bufs[b], gsems[b])

        def step(g, _):
            for b in range(NB):
                j = g * NB + b
                pltpu.make_async_copy(
                    xs_hbm.at[ridx_v.at[j]], bufs[b], gsems[b]).wait()
                pltpu.sync_copy(bufs[b], acc_sh.at[cidx_v.at[j]], add=True)

                @pl.when(j + NB < NCH)
                def _():
                    pltpu.async_copy(xs_hbm.at[ridx_v.at[j + NB]],
                                     bufs[b], gsems[b])

            return 0

        lax.fori_loop(0, NCH // NB, step, 0)
        plsc.subcore_barrier()

        def flush(t, _):
            j = s + t * _NS

            @pl.when(j < RCH)
            def _():
                pltpu.sync_copy(acc_sh.at[pl.ds(j * C, C)], bufs[0])
                pltpu.sync_copy(bufs[0], out_hbm.at[pl.ds(c * _N + j * C, C)])

            return 0

        lax.fori_loop(0, RT, flush, 0)

    return pl.kernel(
        body,
        out_type=jax.ShapeDtypeStruct((_NC * _N, D), jnp.float32),
        mesh=_sc_mesh(),
        scratch_types=[
            pltpu.VMEM((NCH, C), jnp.int32),
            pltpu.VMEM((NCH, C), jnp.int32),
        ] + [pltpu.VMEM((C, D), jnp.float32)] * NB + [
            pltpu.VMEM_SHARED((_N, D), jnp.float32),
        ] + [pltpu.SemaphoreType.DMA] * NB,
        compiler_params=_SC_PARAMS,
    )


def _tc_first(x, W, degp):
    def body(x_ref, w_ref, d_ref, xs_ref, dis_ref):
        deg = d_ref[:_N, :] + d_ref[_N:, :] + 1.0
        dis = lax.rsqrt(deg)
        dis_ref[...] = dis
        xs_ref[...] = jnp.dot(x_ref[...], w_ref[...],
                              preferred_element_type=jnp.float32) * dis

    return pl.pallas_call(
        body,
        out_shape=(
            jax.ShapeDtypeStruct((_N, W.shape[1]), jnp.float32),
            jax.ShapeDtypeStruct((_N, 1), jnp.float32),
        ),
    )(x, W, degp)


def _tc_fuse(scat, xs, dis, b, W):
    def body(sc_ref, xs_ref, dis_ref, b_ref, w_ref, o_ref):
        h = sc_ref[:_N, :] + sc_ref[_N:, :] + xs_ref[...]
        h = jnp.maximum(h * dis_ref[...] + b_ref[...], 0.0)
        o_ref[...] = jnp.dot(h, w_ref[...],
                             preferred_element_type=jnp.float32) * dis_ref[...]

    return pl.pallas_call(
        body,
        out_shape=jax.ShapeDtypeStruct((_N, W.shape[1]), jnp.float32),
    )(scat, xs, dis, b, W)


def _tc_final(scat, xs, dis, b, batch_row, Wfc, bfc):
    def body(sc_ref, xs_ref, dis_ref, b_ref, bt_ref, wfc_ref, bfc_ref, o_ref):
        h = sc_ref[:_N, :] + sc_ref[_N:, :] + xs_ref[...]
        h = jnp.maximum(h * dis_ref[...] + b_ref[...], 0.0)
        gids = lax.broadcasted_iota(jnp.int32, (_G, _N), 0)
        m = (bt_ref[...] == gids).astype(jnp.float32)
        sums = jnp.dot(m, h, preferred_element_type=jnp.float32)
        cnt = jnp.maximum(jnp.sum(m, axis=1), 1.0)
        pooled = sums / cnt[:, None]
        o_ref[...] = jnp.dot(pooled, wfc_ref[...],
                             preferred_element_type=jnp.float32) + bfc_ref[...]

    return pl.pallas_call(
        body,
        out_shape=jax.ShapeDtypeStruct((_G, 1), jnp.float32),
    )(scat, xs, dis, b, batch_row, Wfc, bfc)


def kernel(x, edge_index, batch, W1, b1, W2, b2, W3, b3, W4, b4, Wfc, bfc):
    ei = edge_index.astype(jnp.int32)
    row40 = ei[0].reshape(_E // 40, 40)
    col40 = ei[1].reshape(_E // 40, 40)
    row80 = ei[0].reshape(_E // 80, 80)
    col80 = ei[1].reshape(_E // 80, 80)

    degp = _make_degree_kernel()(col80)
    xs1, dis = _tc_first(x, W1, degp.reshape(_NC * _N, 1))

    sc1 = _make_scatter_kernel(128, 40, 5)(xs1, row40, col40)
    xs2 = _tc_fuse(sc1, xs1, dis, b1.reshape(1, -1), W2)

    sc2 = _make_scatter_kernel(64, 80, 5)(xs2, row80, col80)
    xs3 = _tc_fuse(sc2, xs2, dis, b2.reshape(1, -1), W3)

    sc3 = _make_scatter_kernel(64, 80, 5)(xs3, row80, col80)
    xs4 = _tc_fuse(sc3, xs3, dis, b3.reshape(1, -1), W4)

    sc4 = _make_scatter_kernel(32, 80, 5)(xs4, row80, col80)
    return _tc_final(sc4, xs4, dis, b4.reshape(1, -1),
                     batch.reshape(1, _N).astype(jnp.int32),
                     Wfc, bfc.reshape(1, 1))
